# baseline (speedup 1.0000x reference)
import jax
import jax.numpy as jnp
from jax.experimental import pallas as pl

N = 10000
C = 100
NU = 32
B = 16
F = 1 + NU


def _mm_body(x_ref, w_ref, b_ref, o_ref):
    x = x_ref[...]
    o_ref[...] = x @ w_ref[...] + b_ref[...]


def _out_matmul(x1, W, b):
    out = W.shape[1]
    blk = 1000
    return pl.pallas_call(
        _mm_body,
        grid=(N // blk,),
        in_specs=[
            pl.BlockSpec((blk * B, F), lambda i: (i, 0)),
            pl.BlockSpec((F, out), lambda i: (0, 0)),
            pl.BlockSpec((out,), lambda i: (0,)),
        ],
        out_specs=pl.BlockSpec((blk * B, out), lambda i: (i, 0)),
        out_shape=jax.ShapeDtypeStruct((N * B, out), jnp.float32),
    )(x1.reshape(N * B, F), W, b)


def _spmm(row, col, val, x):
    return jax.ops.segment_sum(val[:, None] * x[col], row, num_segments=N)


def _gc_live(x0, W, b, afc, L_val, L_row, L_col):
    x0fc = afc.T @ x0
    g = afc @ jax.nn.sigmoid(x0fc)
    y = _spmm(L_row, L_col, L_val, x0)
    x1 = y + g
    return _out_matmul(x1.reshape(N, B, F), W, b)


def kernel(inputs, state, weights_0, bias_0, weights_1, bias_1, weights_01,
           bias_01, weights_11, bias_11, afc_mx, L_val, L1, L_row, L_col):
    st = state.reshape(B, N, NU).transpose(1, 0, 2)
    xin = inputs.T.reshape(N, B, 1)
    x0 = jnp.concatenate([xin, st], axis=2).reshape(N, B * F)

    out1 = _gc_live(x0, weights_0, bias_0, afc_mx, L_val, L_row, L_col)
    value = jax.nn.sigmoid(out1).reshape(N, B, 2 * NU)
    r = value[:, :, :NU]
    u = value[:, :, NU:]

    st2 = r * st
    x0b = jnp.concatenate([xin, st2], axis=2).reshape(N, B * F)
    out2 = _gc_live(x0b, weights_1, bias_1, afc_mx, L_val, L_row, L_col)
    c = jnp.tanh(out2).reshape(N, B, NU)

    ns = u * st + (1.0 - u) * c
    return ns.transpose(1, 0, 2).reshape(B, N * NU)


# speedup vs baseline: 1.9652x; 1.9652x over previous
import functools

import jax
import jax.numpy as jnp
from jax import lax
from jax.experimental import pallas as pl
from jax.experimental.pallas import tpu as pltpu
from jax.experimental.pallas import tpu_sc as plsc

N = 10000
C = 100
NU = 32
B = 16
F = 1 + NU
W528 = B * F
WPAD = 576
NF = 18
FPP = 6
NPASS = NF // FPP
KE = 2048
NTILES = 32


def _sc_spmm(xT, crp, valp):
    epad = crp.shape[0]
    nblk = epad // KE

    mesh = plsc.VectorSubcoreMesh(core_axis_name="c", subcore_axis_name="s")

    @functools.partial(
        pl.kernel,
        out_type=jax.ShapeDtypeStruct((WPAD, N), jnp.float32),
        mesh=mesh,
        compiler_params=pltpu.CompilerParams(
            use_tc_tiling_on_sc=False, needs_layout_passes=False),
        scratch_types=(
            [pltpu.VMEM((N,), jnp.float32) for _ in range(2 * FPP)]
            + [pltpu.VMEM((KE,), jnp.int32),
               pltpu.VMEM((KE,), jnp.float32)]
        ),
    )
    def k(xT_hbm, cr_hbm, val_hbm, out_hbm, *bufs):
        xb = bufs[:FPP]
        yb = bufs[FPP:2 * FPP]
        crb = bufs[2 * FPP]
        vb = bufs[2 * FPP + 1]
        c = lax.axis_index("c")
        s = lax.axis_index("s")
        fbase = (c * 16 + s) * NF
        zv = jnp.zeros((16,), jnp.float32)

        for p in range(NPASS):
            f0 = fbase + p * FPP
            for j in range(FPP):
                pltpu.sync_copy(xT_hbm.at[f0 + j], xb[j])

            def zrow(i, _):
                for j in range(FPP):
                    yb[j][pl.ds(i * 16, 16)] = zv
                return 0

            lax.fori_loop(0, N // 16, zrow, 0)

            def blk(b, _):
                eb = b * KE
                pltpu.sync_copy(cr_hbm.at[pl.ds(eb, KE)], crb)
                pltpu.sync_copy(val_hbm.at[pl.ds(eb, KE)], vb)

                def grp(g, _):
                    crv = crb[pl.ds(g * 16, 16)]
                    vv = vb[pl.ds(g * 16, 16)]
                    colv = lax.shift_right_logical(crv, 14)
                    rowv = lax.bitwise_and(crv, 16383)
                    for j in range(FPP):
                        xv = plsc.load_gather(xb[j], [colv])
                        plsc.addupdate_scatter(yb[j], [rowv], xv * vv)
                    return 0

                lax.fori_loop(0, KE // 16, grp, 0)
                return 0

            lax.fori_loop(0, nblk, blk, 0)
            for j in range(FPP):
                pltpu.sync_copy(yb[j], out_hbm.at[f0 + j])

    return k(xT, crp, valp)


BLK = 1000


def _sfc_body(afc_ref, x0_ref, o_ref):
    i = pl.program_id(0)

    @pl.when(i == 0)
    def _():
        o_ref[...] = jnp.zeros_like(o_ref)

    o_ref[...] += lax.dot_general(
        afc_ref[...], x0_ref[...], (((0,), (0,)), ((), ())),
        preferred_element_type=jnp.float32)

    @pl.when(i == pl.num_programs(0) - 1)
    def _():
        o_ref[...] = jax.nn.sigmoid(o_ref[...])


def _sfc(afc, x0):
    return pl.pallas_call(
        _sfc_body,
        grid=(N // BLK,),
        in_specs=[
            pl.BlockSpec((BLK, C), lambda i: (i, 0)),
            pl.BlockSpec((BLK, WPAD), lambda i: (i, 0)),
        ],
        out_specs=pl.BlockSpec((C, WPAD), lambda i: (0, 0)),
        out_shape=jax.ShapeDtypeStruct((C, WPAD), jnp.float32),
    )(afc, x0)


def _x1_block(y_ref, afc_ref, sfc_ref):
    g = jnp.dot(afc_ref[...], sfc_ref[...], preferred_element_type=jnp.float32)
    return y_ref[...] + g


def _round1_body(y_ref, afc_ref, sfc_ref, xin_ref, stn_ref,
                 wexp_ref, brr_ref, bru_ref, pin_ref, pst_ref,
                 u_ref, x0b_ref):
    x1 = _x1_block(y_ref, afc_ref, sfc_ref)
    ru = jax.nn.sigmoid(
        jnp.dot(x1, wexp_ref[...], preferred_element_type=jnp.float32)
        + jnp.concatenate([brr_ref[...], bru_ref[...]], axis=-1))
    r = ru[:, :B * NU]
    u_ref[...] = ru[:, B * NU:]
    st2 = r * stn_ref[...]
    x0b_ref[...] = (
        jnp.dot(xin_ref[...], pin_ref[...], preferred_element_type=jnp.float32)
        + jnp.dot(st2, pst_ref[...], preferred_element_type=jnp.float32))


def _round2_body(y_ref, afc_ref, sfc_ref, stn_ref, u_ref,
                 wexp_ref, bt_ref, ns_ref):
    x1 = _x1_block(y_ref, afc_ref, sfc_ref)
    out2 = (jnp.dot(x1, wexp_ref[...], preferred_element_type=jnp.float32)
            + bt_ref[...])
    cc = jnp.tanh(out2)
    u = u_ref[...]
    ns_ref[...] = u * stn_ref[...] + (1.0 - u) * cc


def _round1(y, afc, sfc, xin, stn, wexp, brr, bru, pin, pst):
    return pl.pallas_call(
        _round1_body,
        grid=(N // BLK,),
        in_specs=[
            pl.BlockSpec((BLK, WPAD), lambda i: (i, 0)),
            pl.BlockSpec((BLK, C), lambda i: (i, 0)),
            pl.BlockSpec((C, WPAD), lambda i: (0, 0)),
            pl.BlockSpec((BLK, B), lambda i: (i, 0)),
            pl.BlockSpec((BLK, B * NU), lambda i: (i, 0)),
            pl.BlockSpec((WPAD, 2 * B * NU), lambda i: (0, 0)),
            pl.BlockSpec((B * NU,), lambda i: (0,)),
            pl.BlockSpec((B * NU,), lambda i: (0,)),
            pl.BlockSpec((B, WPAD), lambda i: (0, 0)),
            pl.BlockSpec((B * NU, WPAD), lambda i: (0, 0)),
        ],
        out_specs=[
            pl.BlockSpec((BLK, B * NU), lambda i: (i, 0)),
            pl.BlockSpec((BLK, WPAD), lambda i: (i, 0)),
        ],
        out_shape=[
            jax.ShapeDtypeStruct((N, B * NU), jnp.float32),
            jax.ShapeDtypeStruct((N, WPAD), jnp.float32),
        ],
    )(y, afc, sfc, xin, stn, wexp, brr, bru, pin, pst)


def _round2(y, afc, sfc, stn, u, wexp, bt):
    return pl.pallas_call(
        _round2_body,
        grid=(N // BLK,),
        in_specs=[
            pl.BlockSpec((BLK, WPAD), lambda i: (i, 0)),
            pl.BlockSpec((BLK, C), lambda i: (i, 0)),
            pl.BlockSpec((C, WPAD), lambda i: (0, 0)),
            pl.BlockSpec((BLK, B * NU), lambda i: (i, 0)),
            pl.BlockSpec((BLK, B * NU), lambda i: (i, 0)),
            pl.BlockSpec((WPAD, B * NU), lambda i: (0, 0)),
            pl.BlockSpec((B * NU,), lambda i: (0,)),
        ],
        out_specs=pl.BlockSpec((BLK, B * NU), lambda i: (i, 0)),
        out_shape=jax.ShapeDtypeStruct((N, B * NU), jnp.float32),
    )(y, afc, sfc, stn, u, wexp, bt)


def kernel(inputs, state, weights_0, bias_0, weights_1, bias_1, weights_01,
           bias_01, weights_11, bias_11, afc_mx, L_val, L1, L_row, L_col):
    stn3 = state.reshape(B, N, NU).transpose(1, 0, 2)
    stn = stn3.reshape(N, B * NU)
    xin = inputs.T
    x0 = jnp.concatenate([xin[:, :, None], stn3], axis=2).reshape(N, W528)
    x0 = jnp.pad(x0, ((0, 0), (0, WPAD - W528)))

    e = L_val.shape[0]
    epad = ((e + KE - 1) // KE) * KE
    pad = epad - e
    crp = jnp.pad(L_col * 16384 + L_row, (0, pad))
    valp = jnp.pad(L_val, (0, pad))

    eyeb = jnp.eye(B, dtype=jnp.float32)
    zpad = jnp.zeros((WPAD - W528, 2 * B * NU), jnp.float32)
    wexp0 = jnp.concatenate([
        jnp.concatenate(
            [jnp.kron(eyeb, weights_0[:, :NU]),
             jnp.kron(eyeb, weights_0[:, NU:])], axis=1),
        zpad], axis=0)
    wexp1 = jnp.concatenate(
        [jnp.kron(eyeb, weights_1), zpad[:, :B * NU]], axis=0)
    brr = jnp.tile(bias_0[:NU], B)
    bru = jnp.tile(bias_0[NU:], B)
    bt1 = jnp.tile(bias_1, B)
    pin = jnp.zeros((B, WPAD), jnp.float32)
    pin = pin.at[jnp.arange(B), jnp.arange(B) * F].set(1.0)
    pst = jnp.zeros((B * NU, WPAD), jnp.float32)
    src = jnp.arange(B * NU)
    pst = pst.at[src, (src // NU) * F + 1 + (src % NU)].set(1.0)

    sfc1 = _sfc(afc_mx, x0)
    y1 = _sc_spmm(x0.T, crp, valp).T
    u, x0b = _round1(y1, afc_mx, sfc1, xin, stn, wexp0, brr, bru, pin, pst)

    sfc2 = _sfc(afc_mx, x0b)
    y2 = _sc_spmm(x0b.T, crp, valp).T
    ns = _round2(y2, afc_mx, sfc2, stn, u, wexp1, bt1)

    return ns.reshape(N, B, NU).transpose(1, 0, 2).reshape(B, N * NU)


# speedup vs baseline: 2.2942x; 1.1674x over previous
import functools

import jax
import jax.numpy as jnp
from jax import lax
from jax.experimental import pallas as pl
from jax.experimental.pallas import tpu as pltpu
from jax.experimental.pallas import tpu_sc as plsc

N = 10000
C = 100
NU = 32
B = 16
F = 1 + NU
W528 = B * F
WPAD = 576
NF = 18
FPP = 6
NPASS = NF // FPP
KE = 1792
NBUF = 2
NTILES = 32


def _sc_spmm(xT, crp, valp):
    epad = crp.shape[0]
    nblk = epad // KE

    mesh = plsc.VectorSubcoreMesh(core_axis_name="c", subcore_axis_name="s")

    @functools.partial(
        pl.kernel,
        out_type=jax.ShapeDtypeStruct((WPAD, N), jnp.float32),
        mesh=mesh,
        compiler_params=pltpu.CompilerParams(
            use_tc_tiling_on_sc=False, needs_layout_passes=False),
        scratch_types=(
            [pltpu.VMEM((N,), jnp.float32) for _ in range(2 * FPP)]
            + [pltpu.VMEM((KE,), jnp.int32) for _ in range(NBUF)]
            + [pltpu.VMEM((KE,), jnp.float32) for _ in range(NBUF)]
            + [pltpu.SemaphoreType.DMA for _ in range(2 * NBUF)]
        ),
    )
    def k(xT_hbm, cr_hbm, val_hbm, out_hbm, *bufs):
        xb = bufs[:FPP]
        yb = bufs[FPP:2 * FPP]
        crbs = bufs[2 * FPP:2 * FPP + NBUF]
        vbs = bufs[2 * FPP + NBUF:2 * FPP + 2 * NBUF]
        csem = bufs[2 * FPP + 2 * NBUF:2 * FPP + 3 * NBUF]
        vsem = bufs[2 * FPP + 3 * NBUF:]
        c = lax.axis_index("c")
        s = lax.axis_index("s")
        fbase = (c * 16 + s) * NF
        zv = jnp.zeros((16,), jnp.float32)

        def edge_wait(i):
            pltpu.make_async_copy(
                cr_hbm.at[pl.ds(0, KE)], crbs[i], csem[i]).wait()
            pltpu.make_async_copy(
                val_hbm.at[pl.ds(0, KE)], vbs[i], vsem[i]).wait()

        def edge_fetch(i, b):
            eb = b * KE
            pltpu.async_copy(cr_hbm.at[pl.ds(eb, KE)], crbs[i], csem[i])
            pltpu.async_copy(val_hbm.at[pl.ds(eb, KE)], vbs[i], vsem[i])

        for p in range(NPASS):
            f0 = fbase + p * FPP
            for j in range(FPP):
                pltpu.sync_copy(xT_hbm.at[f0 + j], xb[j])

            def zrow(i, _):
                for j in range(FPP):
                    yb[j][pl.ds(i * 16, 16)] = zv
                return 0

            lax.fori_loop(0, N // 16, zrow, 0)

            for i in range(NBUF):
                edge_fetch(i, jnp.int32(i))

            def outer(ob, _):
                for i in range(NBUF):
                    b = ob * NBUF + i
                    edge_wait(i)
                    crb, vb = crbs[i], vbs[i]

                    def grp(gg, _):
                        for h in range(2):
                            g = gg * 2 + h
                            crv = crb[pl.ds(g * 16, 16)]
                            vv = vb[pl.ds(g * 16, 16)]
                            colv = lax.shift_right_logical(crv, 14)
                            rowv = lax.bitwise_and(crv, 16383)
                            for j in range(FPP):
                                xv = plsc.load_gather(xb[j], [colv])
                                plsc.addupdate_scatter(yb[j], [rowv], xv * vv)
                        return 0

                    lax.fori_loop(0, KE // 32, grp, 0)
                    edge_fetch(i, jnp.minimum(b + NBUF, nblk - 1))
                return 0

            lax.fori_loop(0, nblk // NBUF, outer, 0)
            for i in range(NBUF):
                edge_wait(i)
            for j in range(FPP):
                pltpu.sync_copy(yb[j], out_hbm.at[f0 + j])

    return k(xT, crp, valp)


BLK = 1000


def _sfc_body(afc_ref, x0_ref, o_ref):
    i = pl.program_id(0)

    @pl.when(i == 0)
    def _():
        o_ref[...] = jnp.zeros_like(o_ref)

    o_ref[...] += lax.dot_general(
        afc_ref[...], x0_ref[...], (((0,), (0,)), ((), ())),
        preferred_element_type=jnp.float32)

    @pl.when(i == pl.num_programs(0) - 1)
    def _():
        o_ref[...] = jax.nn.sigmoid(o_ref[...])


def _sfc(afc, x0):
    return pl.pallas_call(
        _sfc_body,
        grid=(N // BLK,),
        in_specs=[
            pl.BlockSpec((BLK, C), lambda i: (i, 0)),
            pl.BlockSpec((BLK, WPAD), lambda i: (i, 0)),
        ],
        out_specs=pl.BlockSpec((C, WPAD), lambda i: (0, 0)),
        out_shape=jax.ShapeDtypeStruct((C, WPAD), jnp.float32),
    )(afc, x0)


def _x1_block(y_ref, afc_ref, sfc_ref):
    g = jnp.dot(afc_ref[...], sfc_ref[...], preferred_element_type=jnp.float32)
    return y_ref[...] + g


def _round1_body(y_ref, afc_ref, sfc_ref, xin_ref, stn_ref,
                 wexp_ref, brr_ref, bru_ref, pin_ref, pst_ref,
                 u_ref, x0b_ref):
    x1 = _x1_block(y_ref, afc_ref, sfc_ref)
    ru = jax.nn.sigmoid(
        jnp.dot(x1, wexp_ref[...], preferred_element_type=jnp.float32)
        + jnp.concatenate([brr_ref[...], bru_ref[...]], axis=-1))
    r = ru[:, :B * NU]
    u_ref[...] = ru[:, B * NU:]
    st2 = r * stn_ref[...]
    x0b_ref[...] = (
        jnp.dot(xin_ref[...], pin_ref[...], preferred_element_type=jnp.float32)
        + jnp.dot(st2, pst_ref[...], preferred_element_type=jnp.float32))


def _round2_body(y_ref, afc_ref, sfc_ref, stn_ref, u_ref,
                 wexp_ref, bt_ref, ns_ref):
    x1 = _x1_block(y_ref, afc_ref, sfc_ref)
    out2 = (jnp.dot(x1, wexp_ref[...], preferred_element_type=jnp.float32)
            + bt_ref[...])
    cc = jnp.tanh(out2)
    u = u_ref[...]
    ns_ref[...] = u * stn_ref[...] + (1.0 - u) * cc


def _round1(y, afc, sfc, xin, stn, wexp, brr, bru, pin, pst):
    return pl.pallas_call(
        _round1_body,
        grid=(N // BLK,),
        in_specs=[
            pl.BlockSpec((BLK, WPAD), lambda i: (i, 0)),
            pl.BlockSpec((BLK, C), lambda i: (i, 0)),
            pl.BlockSpec((C, WPAD), lambda i: (0, 0)),
            pl.BlockSpec((BLK, B), lambda i: (i, 0)),
            pl.BlockSpec((BLK, B * NU), lambda i: (i, 0)),
            pl.BlockSpec((WPAD, 2 * B * NU), lambda i: (0, 0)),
            pl.BlockSpec((B * NU,), lambda i: (0,)),
            pl.BlockSpec((B * NU,), lambda i: (0,)),
            pl.BlockSpec((B, WPAD), lambda i: (0, 0)),
            pl.BlockSpec((B * NU, WPAD), lambda i: (0, 0)),
        ],
        out_specs=[
            pl.BlockSpec((BLK, B * NU), lambda i: (i, 0)),
            pl.BlockSpec((BLK, WPAD), lambda i: (i, 0)),
        ],
        out_shape=[
            jax.ShapeDtypeStruct((N, B * NU), jnp.float32),
            jax.ShapeDtypeStruct((N, WPAD), jnp.float32),
        ],
    )(y, afc, sfc, xin, stn, wexp, brr, bru, pin, pst)


def _round2(y, afc, sfc, stn, u, wexp, bt):
    return pl.pallas_call(
        _round2_body,
        grid=(N // BLK,),
        in_specs=[
            pl.BlockSpec((BLK, WPAD), lambda i: (i, 0)),
            pl.BlockSpec((BLK, C), lambda i: (i, 0)),
            pl.BlockSpec((C, WPAD), lambda i: (0, 0)),
            pl.BlockSpec((BLK, B * NU), lambda i: (i, 0)),
            pl.BlockSpec((BLK, B * NU), lambda i: (i, 0)),
            pl.BlockSpec((WPAD, B * NU), lambda i: (0, 0)),
            pl.BlockSpec((B * NU,), lambda i: (0,)),
        ],
        out_specs=pl.BlockSpec((BLK, B * NU), lambda i: (i, 0)),
        out_shape=jax.ShapeDtypeStruct((N, B * NU), jnp.float32),
    )(y, afc, sfc, stn, u, wexp, bt)


def kernel(inputs, state, weights_0, bias_0, weights_1, bias_1, weights_01,
           bias_01, weights_11, bias_11, afc_mx, L_val, L1, L_row, L_col):
    stn3 = state.reshape(B, N, NU).transpose(1, 0, 2)
    stn = stn3.reshape(N, B * NU)
    xin = inputs.T
    x0 = jnp.concatenate([xin[:, :, None], stn3], axis=2).reshape(N, W528)
    x0 = jnp.pad(x0, ((0, 0), (0, WPAD - W528)))

    e = L_val.shape[0]
    epad = ((e + NBUF * KE - 1) // (NBUF * KE)) * (NBUF * KE)
    pad = epad - e
    crp = jnp.pad(L_col * 16384 + L_row, (0, pad))
    valp = jnp.pad(L_val, (0, pad))

    eyeb = jnp.eye(B, dtype=jnp.float32)
    zpad = jnp.zeros((WPAD - W528, 2 * B * NU), jnp.float32)
    wexp0 = jnp.concatenate([
        jnp.concatenate(
            [jnp.kron(eyeb, weights_0[:, :NU]),
             jnp.kron(eyeb, weights_0[:, NU:])], axis=1),
        zpad], axis=0)
    wexp1 = jnp.concatenate(
        [jnp.kron(eyeb, weights_1), zpad[:, :B * NU]], axis=0)
    brr = jnp.tile(bias_0[:NU], B)
    bru = jnp.tile(bias_0[NU:], B)
    bt1 = jnp.tile(bias_1, B)
    pin = jnp.zeros((B, WPAD), jnp.float32)
    pin = pin.at[jnp.arange(B), jnp.arange(B) * F].set(1.0)
    pst = jnp.zeros((B * NU, WPAD), jnp.float32)
    src = jnp.arange(B * NU)
    pst = pst.at[src, (src // NU) * F + 1 + (src % NU)].set(1.0)

    sfc1 = _sfc(afc_mx, x0)
    y1 = _sc_spmm(x0.T, crp, valp).T
    u, x0b = _round1(y1, afc_mx, sfc1, xin, stn, wexp0, brr, bru, pin, pst)

    sfc2 = _sfc(afc_mx, x0b)
    y2 = _sc_spmm(x0b.T, crp, valp).T
    ns = _round2(y2, afc_mx, sfc2, stn, u, wexp1, bt1)

    return ns.reshape(N, B, NU).transpose(1, 0, 2).reshape(B, N * NU)


# speedup vs baseline: 4.2286x; 1.8432x over previous
import functools

import jax
import jax.numpy as jnp
from jax import lax
from jax.experimental import pallas as pl
from jax.experimental.pallas import tpu as pltpu
from jax.experimental.pallas import tpu_sc as plsc

N = 10000
C = 100
NU = 32
B = 16
F = 1 + NU
W528 = B * F
WPAD = 576
NF = 18
FPP = 6
NPASS = NF // FPP
KE = 1792
NBUF = 2
NTILES = 32


def _sc_spmm(xT, crp, valp):
    epad = crp.shape[0]
    nblk = epad // KE

    mesh = plsc.VectorSubcoreMesh(core_axis_name="c", subcore_axis_name="s")

    @functools.partial(
        pl.kernel,
        out_type=jax.ShapeDtypeStruct((WPAD, N), jnp.float32),
        mesh=mesh,
        compiler_params=pltpu.CompilerParams(
            use_tc_tiling_on_sc=False, needs_layout_passes=False),
        scratch_types=(
            [pltpu.VMEM((N,), jnp.float32) for _ in range(2 * FPP)]
            + [pltpu.VMEM((KE,), jnp.int32) for _ in range(NBUF)]
            + [pltpu.VMEM((KE,), jnp.float32) for _ in range(NBUF)]
            + [pltpu.SemaphoreType.DMA for _ in range(2 * NBUF)]
        ),
    )
    def k(xT_hbm, cr_hbm, val_hbm, out_hbm, *bufs):
        xb = bufs[:FPP]
        yb = bufs[FPP:2 * FPP]
        crbs = bufs[2 * FPP:2 * FPP + NBUF]
        vbs = bufs[2 * FPP + NBUF:2 * FPP + 2 * NBUF]
        csem = bufs[2 * FPP + 2 * NBUF:2 * FPP + 3 * NBUF]
        vsem = bufs[2 * FPP + 3 * NBUF:]
        c = lax.axis_index("c")
        s = lax.axis_index("s")
        fbase = (c * 16 + s) * NF
        zv = jnp.zeros((16,), jnp.float32)

        def edge_wait(i):
            pltpu.make_async_copy(
                cr_hbm.at[pl.ds(0, KE)], crbs[i], csem[i]).wait()
            pltpu.make_async_copy(
                val_hbm.at[pl.ds(0, KE)], vbs[i], vsem[i]).wait()

        def edge_fetch(i, b):
            eb = b * KE
            pltpu.async_copy(cr_hbm.at[pl.ds(eb, KE)], crbs[i], csem[i])
            pltpu.async_copy(val_hbm.at[pl.ds(eb, KE)], vbs[i], vsem[i])

        for p in range(NPASS):
            f0 = fbase + p * FPP
            for j in range(FPP):
                pltpu.sync_copy(xT_hbm.at[f0 + j], xb[j])

            def zrow(i, _):
                for j in range(FPP):
                    yb[j][pl.ds(i * 16, 16)] = zv
                return 0

            lax.fori_loop(0, N // 16, zrow, 0)

            for i in range(NBUF):
                edge_fetch(i, jnp.int32(i))

            def outer(ob, _):
                for i in range(NBUF):
                    b = ob * NBUF + i
                    edge_wait(i)
                    crb, vb = crbs[i], vbs[i]

                    @plsc.parallel_loop(0, KE // 32, 1, unroll=2)
                    def grp(gg):
                        heads = []
                        for h in range(2):
                            g = gg * 2 + h
                            crv = crb[pl.ds(g * 16, 16)]
                            vv = vb[pl.ds(g * 16, 16)]
                            colv = lax.shift_right_logical(crv, 14)
                            rowv = lax.bitwise_and(crv, 16383)
                            heads.append((colv, rowv, vv))
                        gathered = [
                            [plsc.load_gather(xb[j], [colv])
                             for j in range(FPP)]
                            for (colv, _, _) in heads]
                        for h in range(2):
                            _, rowv, vv = heads[h]
                            for j in range(FPP):
                                plsc.addupdate_scatter(
                                    yb[j], [rowv], gathered[h][j] * vv)

                    edge_fetch(i, jnp.minimum(b + NBUF, nblk - 1))
                return 0

            lax.fori_loop(0, nblk // NBUF, outer, 0)
            for i in range(NBUF):
                edge_wait(i)
            for j in range(FPP):
                pltpu.sync_copy(yb[j], out_hbm.at[f0 + j])

    return k(xT, crp, valp)


BLK = 1000


def _sfc_body(afc_ref, x0_ref, o_ref):
    i = pl.program_id(0)

    @pl.when(i == 0)
    def _():
        o_ref[...] = jnp.zeros_like(o_ref)

    o_ref[...] += lax.dot_general(
        afc_ref[...], x0_ref[...], (((0,), (0,)), ((), ())),
        preferred_element_type=jnp.float32)

    @pl.when(i == pl.num_programs(0) - 1)
    def _():
        o_ref[...] = jax.nn.sigmoid(o_ref[...])


def _sfc(afc, x0):
    return pl.pallas_call(
        _sfc_body,
        grid=(N // BLK,),
        in_specs=[
            pl.BlockSpec((BLK, C), lambda i: (i, 0)),
            pl.BlockSpec((BLK, WPAD), lambda i: (i, 0)),
        ],
        out_specs=pl.BlockSpec((C, WPAD), lambda i: (0, 0)),
        out_shape=jax.ShapeDtypeStruct((C, WPAD), jnp.float32),
    )(afc, x0)


def _x1_block(y_ref, afc_ref, sfc_ref):
    g = jnp.dot(afc_ref[...], sfc_ref[...], preferred_element_type=jnp.float32)
    return y_ref[...] + g


def _round1_body(y_ref, afc_ref, sfc_ref, xin_ref, stn_ref,
                 wexp_ref, brr_ref, bru_ref, pin_ref, pst_ref,
                 u_ref, x0b_ref):
    x1 = _x1_block(y_ref, afc_ref, sfc_ref)
    ru = jax.nn.sigmoid(
        jnp.dot(x1, wexp_ref[...], preferred_element_type=jnp.float32)
        + jnp.concatenate([brr_ref[...], bru_ref[...]], axis=-1))
    r = ru[:, :B * NU]
    u_ref[...] = ru[:, B * NU:]
    st2 = r * stn_ref[...]
    x0b_ref[...] = (
        jnp.dot(xin_ref[...], pin_ref[...], preferred_element_type=jnp.float32)
        + jnp.dot(st2, pst_ref[...], preferred_element_type=jnp.float32))


def _round2_body(y_ref, afc_ref, sfc_ref, stn_ref, u_ref,
                 wexp_ref, bt_ref, ns_ref):
    x1 = _x1_block(y_ref, afc_ref, sfc_ref)
    out2 = (jnp.dot(x1, wexp_ref[...], preferred_element_type=jnp.float32)
            + bt_ref[...])
    cc = jnp.tanh(out2)
    u = u_ref[...]
    ns_ref[...] = u * stn_ref[...] + (1.0 - u) * cc


def _round1(y, afc, sfc, xin, stn, wexp, brr, bru, pin, pst):
    return pl.pallas_call(
        _round1_body,
        grid=(N // BLK,),
        in_specs=[
            pl.BlockSpec((BLK, WPAD), lambda i: (i, 0)),
            pl.BlockSpec((BLK, C), lambda i: (i, 0)),
            pl.BlockSpec((C, WPAD), lambda i: (0, 0)),
            pl.BlockSpec((BLK, B), lambda i: (i, 0)),
            pl.BlockSpec((BLK, B * NU), lambda i: (i, 0)),
            pl.BlockSpec((WPAD, 2 * B * NU), lambda i: (0, 0)),
            pl.BlockSpec((B * NU,), lambda i: (0,)),
            pl.BlockSpec((B * NU,), lambda i: (0,)),
            pl.BlockSpec((B, WPAD), lambda i: (0, 0)),
            pl.BlockSpec((B * NU, WPAD), lambda i: (0, 0)),
        ],
        out_specs=[
            pl.BlockSpec((BLK, B * NU), lambda i: (i, 0)),
            pl.BlockSpec((BLK, WPAD), lambda i: (i, 0)),
        ],
        out_shape=[
            jax.ShapeDtypeStruct((N, B * NU), jnp.float32),
            jax.ShapeDtypeStruct((N, WPAD), jnp.float32),
        ],
    )(y, afc, sfc, xin, stn, wexp, brr, bru, pin, pst)


def _round2(y, afc, sfc, stn, u, wexp, bt):
    return pl.pallas_call(
        _round2_body,
        grid=(N // BLK,),
        in_specs=[
            pl.BlockSpec((BLK, WPAD), lambda i: (i, 0)),
            pl.BlockSpec((BLK, C), lambda i: (i, 0)),
            pl.BlockSpec((C, WPAD), lambda i: (0, 0)),
            pl.BlockSpec((BLK, B * NU), lambda i: (i, 0)),
            pl.BlockSpec((BLK, B * NU), lambda i: (i, 0)),
            pl.BlockSpec((WPAD, B * NU), lambda i: (0, 0)),
            pl.BlockSpec((B * NU,), lambda i: (0,)),
        ],
        out_specs=pl.BlockSpec((BLK, B * NU), lambda i: (i, 0)),
        out_shape=jax.ShapeDtypeStruct((N, B * NU), jnp.float32),
    )(y, afc, sfc, stn, u, wexp, bt)


def kernel(inputs, state, weights_0, bias_0, weights_1, bias_1, weights_01,
           bias_01, weights_11, bias_11, afc_mx, L_val, L1, L_row, L_col):
    stn3 = state.reshape(B, N, NU).transpose(1, 0, 2)
    stn = stn3.reshape(N, B * NU)
    xin = inputs.T
    x0 = jnp.concatenate([xin[:, :, None], stn3], axis=2).reshape(N, W528)
    x0 = jnp.pad(x0, ((0, 0), (0, WPAD - W528)))

    e = L_val.shape[0]
    epad = ((e + NBUF * KE - 1) // (NBUF * KE)) * (NBUF * KE)
    pad = epad - e
    crp = jnp.pad(L_col * 16384 + L_row, (0, pad))
    valp = jnp.pad(L_val, (0, pad))

    eyeb = jnp.eye(B, dtype=jnp.float32)
    zpad = jnp.zeros((WPAD - W528, 2 * B * NU), jnp.float32)
    wexp0 = jnp.concatenate([
        jnp.concatenate(
            [jnp.kron(eyeb, weights_0[:, :NU]),
             jnp.kron(eyeb, weights_0[:, NU:])], axis=1),
        zpad], axis=0)
    wexp1 = jnp.concatenate(
        [jnp.kron(eyeb, weights_1), zpad[:, :B * NU]], axis=0)
    brr = jnp.tile(bias_0[:NU], B)
    bru = jnp.tile(bias_0[NU:], B)
    bt1 = jnp.tile(bias_1, B)
    pin = jnp.zeros((B, WPAD), jnp.float32)
    pin = pin.at[jnp.arange(B), jnp.arange(B) * F].set(1.0)
    pst = jnp.zeros((B * NU, WPAD), jnp.float32)
    src = jnp.arange(B * NU)
    pst = pst.at[src, (src // NU) * F + 1 + (src % NU)].set(1.0)

    sfc1 = _sfc(afc_mx, x0)
    y1 = _sc_spmm(x0.T, crp, valp).T
    u, x0b = _round1(y1, afc_mx, sfc1, xin, stn, wexp0, brr, bru, pin, pst)

    sfc2 = _sfc(afc_mx, x0b)
    y2 = _sc_spmm(x0b.T, crp, valp).T
    ns = _round2(y2, afc_mx, sfc2, stn, u, wexp1, bt1)

    return ns.reshape(N, B, NU).transpose(1, 0, 2).reshape(B, N * NU)


# speedup vs baseline: 5.9032x; 1.3960x over previous
import functools

import jax
import jax.numpy as jnp
from jax import lax
from jax.experimental import pallas as pl
from jax.experimental.pallas import tpu as pltpu
from jax.experimental.pallas import tpu_sc as plsc

N = 10000
NPD = 10240
C = 100
NU = 32
B = 16
F = 1 + NU
SW = B * NU
W528 = B * F
WPAD = 576
NF = 18
FPP = 6
NPASS = NF // FPP
KE = 1792
NBUF = 2
NTILES = 32


def _sc_spmm(xT, crp, valp):
    epad = crp.shape[0]
    nblk = epad // KE

    mesh = plsc.VectorSubcoreMesh(core_axis_name="c", subcore_axis_name="s")

    @functools.partial(
        pl.kernel,
        out_type=jax.ShapeDtypeStruct((WPAD, NPD), jnp.float32),
        mesh=mesh,
        compiler_params=pltpu.CompilerParams(
            use_tc_tiling_on_sc=False, needs_layout_passes=False),
        scratch_types=(
            [pltpu.VMEM((NPD,), jnp.float32) for _ in range(2 * FPP)]
            + [pltpu.VMEM((KE,), jnp.int32) for _ in range(NBUF)]
            + [pltpu.VMEM((KE,), jnp.float32) for _ in range(NBUF)]
            + [pltpu.SemaphoreType.DMA for _ in range(2 * NBUF)]
        ),
    )
    def k(xT_hbm, cr_hbm, val_hbm, out_hbm, *bufs):
        xb = bufs[:FPP]
        yb = bufs[FPP:2 * FPP]
        crbs = bufs[2 * FPP:2 * FPP + NBUF]
        vbs = bufs[2 * FPP + NBUF:2 * FPP + 2 * NBUF]
        csem = bufs[2 * FPP + 2 * NBUF:2 * FPP + 3 * NBUF]
        vsem = bufs[2 * FPP + 3 * NBUF:]
        c = lax.axis_index("c")
        s = lax.axis_index("s")
        fbase = (c * 16 + s) * NF
        zv = jnp.zeros((16,), jnp.float32)

        def edge_wait(i):
            pltpu.make_async_copy(
                cr_hbm.at[pl.ds(0, KE)], crbs[i], csem[i]).wait()
            pltpu.make_async_copy(
                val_hbm.at[pl.ds(0, KE)], vbs[i], vsem[i]).wait()

        def edge_fetch(i, b):
            eb = b * KE
            pltpu.async_copy(cr_hbm.at[pl.ds(eb, KE)], crbs[i], csem[i])
            pltpu.async_copy(val_hbm.at[pl.ds(eb, KE)], vbs[i], vsem[i])

        for p in range(NPASS):
            f0 = fbase + p * FPP
            for j in range(FPP):
                pltpu.sync_copy(xT_hbm.at[f0 + j], xb[j])

            def zrow(i, _):
                for j in range(FPP):
                    yb[j][pl.ds(i * 16, 16)] = zv
                return 0

            lax.fori_loop(0, NPD // 16, zrow, 0)

            for i in range(NBUF):
                edge_fetch(i, jnp.int32(i))

            def outer(ob, _):
                for i in range(NBUF):
                    b = ob * NBUF + i
                    edge_wait(i)
                    crb, vb = crbs[i], vbs[i]

                    @plsc.parallel_loop(0, KE // 32, 1, unroll=2)
                    def grp(gg):
                        heads = []
                        for h in range(2):
                            g = gg * 2 + h
                            crv = crb[pl.ds(g * 16, 16)]
                            vv = vb[pl.ds(g * 16, 16)]
                            colv = lax.shift_right_logical(crv, 14)
                            rowv = lax.bitwise_and(crv, 16383)
                            heads.append((colv, rowv, vv))
                        gathered = [
                            [plsc.load_gather(xb[j], [colv])
                             for j in range(FPP)]
                            for (colv, _, _) in heads]
                        for h in range(2):
                            _, rowv, vv = heads[h]
                            for j in range(FPP):
                                plsc.addupdate_scatter(
                                    yb[j], [rowv], gathered[h][j] * vv)

                    edge_fetch(i, jnp.minimum(b + NBUF, nblk - 1))
                return 0

            lax.fori_loop(0, nblk // NBUF, outer, 0)
            for i in range(NBUF):
                edge_wait(i)
            for j in range(FPP):
                pltpu.sync_copy(yb[j], out_hbm.at[f0 + j])

    return k(xT, crp, valp)


BLK = 1024


def _acc_sigmoid(o_ref, contrib):
    i = pl.program_id(0)

    @pl.when(i == 0)
    def _():
        o_ref[...] = jnp.zeros_like(o_ref)

    o_ref[...] += contrib

    @pl.when(i == pl.num_programs(0) - 1)
    def _():
        o_ref[...] = jax.nn.sigmoid(o_ref[...])


def _build_body(inp_ref, stnT_ref, afc_ref, x0T_ref, sfcT_ref):
    x0T = jnp.concatenate(
        [inp_ref[...], stnT_ref[...],
         jnp.zeros((WPAD - W528, BLK), jnp.float32)], axis=0)
    x0T_ref[...] = x0T
    _acc_sigmoid(sfcT_ref, jnp.dot(x0T, afc_ref[...],
                                   preferred_element_type=jnp.float32))


def _build(inp, stnT, afc):
    return pl.pallas_call(
        _build_body,
        grid=(NPD // BLK,),
        in_specs=[
            pl.BlockSpec((B, BLK), lambda i: (0, i)),
            pl.BlockSpec((SW, BLK), lambda i: (0, i)),
            pl.BlockSpec((BLK, C), lambda i: (i, 0)),
        ],
        out_specs=[
            pl.BlockSpec((WPAD, BLK), lambda i: (0, i)),
            pl.BlockSpec((WPAD, C), lambda i: (0, 0)),
        ],
        out_shape=[
            jax.ShapeDtypeStruct((WPAD, NPD), jnp.float32),
            jax.ShapeDtypeStruct((WPAD, C), jnp.float32),
        ],
    )(inp, stnT, afc)


def _x1T(yT_ref, sfcT_ref, afc_ref):
    gT = lax.dot_general(
        sfcT_ref[...], afc_ref[...], (((1,), (1,)), ((), ())),
        preferred_element_type=jnp.float32)
    return yT_ref[...] + gT


def _r1_body(yT_ref, afc_ref, sfcT_ref, inp_ref, stnT_ref,
             wr_ref, wu_ref, brr_ref, bru_ref,
             x0bT_ref, uT_ref, sfc2T_ref):
    x1T = _x1T(yT_ref, sfcT_ref, afc_ref)
    rT = jax.nn.sigmoid(
        lax.dot_general(wr_ref[...], x1T, (((0,), (0,)), ((), ())),
                        preferred_element_type=jnp.float32)
        + brr_ref[...])
    uT_ref[...] = jax.nn.sigmoid(
        lax.dot_general(wu_ref[...], x1T, (((0,), (0,)), ((), ())),
                        preferred_element_type=jnp.float32)
        + bru_ref[...])
    st2T = rT * stnT_ref[...]
    x0bT = jnp.concatenate(
        [inp_ref[...], st2T,
         jnp.zeros((WPAD - W528, BLK), jnp.float32)], axis=0)
    x0bT_ref[...] = x0bT
    _acc_sigmoid(sfc2T_ref, jnp.dot(x0bT, afc_ref[...],
                                    preferred_element_type=jnp.float32))


def _round1(yT, afc, sfcT, inp, stnT, wr, wu, brr, bru):
    return pl.pallas_call(
        _r1_body,
        grid=(NPD // BLK,),
        in_specs=[
            pl.BlockSpec((WPAD, BLK), lambda i: (0, i)),
            pl.BlockSpec((BLK, C), lambda i: (i, 0)),
            pl.BlockSpec((WPAD, C), lambda i: (0, 0)),
            pl.BlockSpec((B, BLK), lambda i: (0, i)),
            pl.BlockSpec((SW, BLK), lambda i: (0, i)),
            pl.BlockSpec((WPAD, SW), lambda i: (0, 0)),
            pl.BlockSpec((WPAD, SW), lambda i: (0, 0)),
            pl.BlockSpec((SW, 1), lambda i: (0, 0)),
            pl.BlockSpec((SW, 1), lambda i: (0, 0)),
        ],
        out_specs=[
            pl.BlockSpec((WPAD, BLK), lambda i: (0, i)),
            pl.BlockSpec((SW, BLK), lambda i: (0, i)),
            pl.BlockSpec((WPAD, C), lambda i: (0, 0)),
        ],
        out_shape=[
            jax.ShapeDtypeStruct((WPAD, NPD), jnp.float32),
            jax.ShapeDtypeStruct((SW, NPD), jnp.float32),
            jax.ShapeDtypeStruct((WPAD, C), jnp.float32),
        ],
    )(yT, afc, sfcT, inp, stnT, wr, wu, brr, bru)


def _r2_body(yT_ref, afc_ref, sfc2T_ref, stnT_ref, uT_ref,
             wc_ref, bt_ref, nsT_ref):
    x1T = _x1T(yT_ref, sfc2T_ref, afc_ref)
    out2T = lax.dot_general(
        wc_ref[...], x1T, (((0,), (0,)), ((), ())),
        preferred_element_type=jnp.float32) + bt_ref[...]
    cT = jnp.tanh(out2T)
    uT = uT_ref[...]
    nsT_ref[...] = uT * stnT_ref[...] + (1.0 - uT) * cT


def _round2(yT, afc, sfc2T, stnT, uT, wc, bt):
    return pl.pallas_call(
        _r2_body,
        grid=(NPD // BLK,),
        in_specs=[
            pl.BlockSpec((WPAD, BLK), lambda i: (0, i)),
            pl.BlockSpec((BLK, C), lambda i: (i, 0)),
            pl.BlockSpec((WPAD, C), lambda i: (0, 0)),
            pl.BlockSpec((SW, BLK), lambda i: (0, i)),
            pl.BlockSpec((SW, BLK), lambda i: (0, i)),
            pl.BlockSpec((WPAD, SW), lambda i: (0, 0)),
            pl.BlockSpec((SW, 1), lambda i: (0, 0)),
        ],
        out_specs=pl.BlockSpec((SW, BLK), lambda i: (0, i)),
        out_shape=jax.ShapeDtypeStruct((SW, NPD), jnp.float32),
    )(yT, afc, sfc2T, stnT, uT, wc, bt)


def kernel(inputs, state, weights_0, bias_0, weights_1, bias_1, weights_01,
           bias_01, weights_11, bias_11, afc_mx, L_val, L1, L_row, L_col):
    stnT = state.reshape(B, N, NU).transpose(0, 2, 1).reshape(SW, N)
    stnT = jnp.pad(stnT, ((0, 0), (0, NPD - N)))
    inp = jnp.pad(inputs, ((0, 0), (0, NPD - N)))
    afc = jnp.pad(afc_mx, ((0, NPD - N), (0, 0)))

    e = L_val.shape[0]
    epad = ((e + NBUF * KE - 1) // (NBUF * KE)) * (NBUF * KE)
    pad = epad - e
    crp = jnp.pad(L_col * 16384 + L_row, (0, pad))
    valp = jnp.pad(L_val, (0, pad))

    eyeb = jnp.eye(B, dtype=jnp.float32)
    zrows = jnp.zeros((WPAD - W528, SW), jnp.float32)

    def wexp(w):
        return jnp.concatenate(
            [jnp.kron(eyeb, w[:1]), jnp.kron(eyeb, w[1:]), zrows], axis=0)

    wr = wexp(weights_0[:, :NU])
    wu = wexp(weights_0[:, NU:])
    wc = wexp(weights_1)
    brr = jnp.tile(bias_0[:NU], B)[:, None]
    bru = jnp.tile(bias_0[NU:], B)[:, None]
    bt1 = jnp.tile(bias_1, B)[:, None]

    x0T, sfcT = _build(inp, stnT, afc)
    yT1 = _sc_spmm(x0T, crp, valp)
    x0bT, uT, sfc2T = _round1(yT1, afc, sfcT, inp, stnT,
                              wr, wu, brr, bru)

    yT2 = _sc_spmm(x0bT, crp, valp)
    nsT = _round2(yT2, afc, sfc2T, stnT, uT, wc, bt1)

    return nsT[:, :N].reshape(B, NU, N).transpose(0, 2, 1).reshape(B, N * NU)


# speedup vs baseline: 6.5780x; 1.1143x over previous
import functools

import jax
import jax.numpy as jnp
from jax import lax
from jax.experimental import pallas as pl
from jax.experimental.pallas import tpu as pltpu
from jax.experimental.pallas import tpu_sc as plsc

N = 10000
NPD = 10240
C = 100
NU = 32
B = 16
F = 1 + NU
SW = B * NU
W528 = B * F
WPAD = 576
NF = 18
FPP = 6
NPASS = NF // FPP
KE = 1792
NBUF = 2
NTILES = 32


def _sc_spmm(xP, crp, valp):
    epad = crp.shape[0]
    nblk = epad // KE
    HALF = WPAD // 2

    mesh = plsc.VectorSubcoreMesh(core_axis_name="c", subcore_axis_name="s")

    @functools.partial(
        pl.kernel,
        out_type=jax.ShapeDtypeStruct((WPAD, NPD), jnp.float32),
        mesh=mesh,
        compiler_params=pltpu.CompilerParams(
            use_tc_tiling_on_sc=False, needs_layout_passes=False),
        scratch_types=(
            [pltpu.VMEM((NPD,), jnp.int32) for _ in range(FPP // 2)]
            + [pltpu.VMEM((NPD,), jnp.float32) for _ in range(FPP)]
            + [pltpu.VMEM((KE,), jnp.int32) for _ in range(NBUF)]
            + [pltpu.VMEM((KE,), jnp.float32) for _ in range(NBUF)]
            + [pltpu.SemaphoreType.DMA for _ in range(2 * NBUF)]
        ),
    )
    def k(xP_hbm, cr_hbm, val_hbm, out_hbm, *bufs):
        npk = FPP // 2
        xpb = bufs[:npk]
        yb = bufs[npk:npk + FPP]
        crbs = bufs[npk + FPP:npk + FPP + NBUF]
        vbs = bufs[npk + FPP + NBUF:npk + FPP + 2 * NBUF]
        csem = bufs[npk + FPP + 2 * NBUF:npk + FPP + 3 * NBUF]
        vsem = bufs[npk + FPP + 3 * NBUF:]
        c = lax.axis_index("c")
        s = lax.axis_index("s")
        pbase = (c * 16 + s) * (NF // 2)
        zv = jnp.zeros((16,), jnp.float32)

        def edge_wait(i):
            pltpu.make_async_copy(
                cr_hbm.at[pl.ds(0, KE)], crbs[i], csem[i]).wait()
            pltpu.make_async_copy(
                val_hbm.at[pl.ds(0, KE)], vbs[i], vsem[i]).wait()

        def edge_fetch(i, b):
            eb = b * KE
            pltpu.async_copy(cr_hbm.at[pl.ds(eb, KE)], crbs[i], csem[i])
            pltpu.async_copy(val_hbm.at[pl.ds(eb, KE)], vbs[i], vsem[i])

        for p in range(NPASS):
            p0 = pbase + p * npk
            for j in range(npk):
                pltpu.sync_copy(xP_hbm.at[p0 + j], xpb[j])

            def zrow(i, _):
                for j in range(FPP):
                    yb[j][pl.ds(i * 16, 16)] = zv
                return 0

            lax.fori_loop(0, NPD // 16, zrow, 0)

            for i in range(NBUF):
                edge_fetch(i, jnp.int32(i))

            def outer(ob, _):
                for i in range(NBUF):
                    b = ob * NBUF + i
                    edge_wait(i)
                    crb, vb = crbs[i], vbs[i]

                    @plsc.parallel_loop(0, KE // 32, 1, unroll=2)
                    def grp(gg):
                        heads = []
                        for h in range(2):
                            g = gg * 2 + h
                            crv = crb[pl.ds(g * 16, 16)]
                            vv = vb[pl.ds(g * 16, 16)]
                            colv = lax.shift_right_logical(crv, 14)
                            rowv = lax.bitwise_and(crv, 16383)
                            heads.append((colv, rowv, vv))
                        gathered = [
                            [plsc.load_gather(xpb[j], [colv])
                             for j in range(npk)]
                            for (colv, _, _) in heads]
                        for h in range(2):
                            _, rowv, vv = heads[h]
                            for j in range(npk):
                                gp = gathered[h][j]
                                fhi = plsc.bitcast(
                                    lax.bitwise_and(gp, -65536), jnp.float32)
                                flo = plsc.bitcast(
                                    lax.shift_left(gp, 16), jnp.float32)
                                plsc.addupdate_scatter(
                                    yb[2 * j], [rowv], fhi * vv)
                                plsc.addupdate_scatter(
                                    yb[2 * j + 1], [rowv], flo * vv)

                    edge_fetch(i, jnp.minimum(b + NBUF, nblk - 1))
                return 0

            lax.fori_loop(0, nblk // NBUF, outer, 0)
            for i in range(NBUF):
                edge_wait(i)
            for j in range(npk):
                pltpu.sync_copy(yb[2 * j], out_hbm.at[p0 + j])
                pltpu.sync_copy(yb[2 * j + 1], out_hbm.at[p0 + j + HALF])

    return k(xP, crp, valp)


BLK = 1024


def _acc_sigmoid(o_ref, contrib):
    i = pl.program_id(0)

    @pl.when(i == 0)
    def _():
        o_ref[...] = jnp.zeros_like(o_ref)

    o_ref[...] += contrib

    @pl.when(i == pl.num_programs(0) - 1)
    def _():
        o_ref[...] = jax.nn.sigmoid(o_ref[...])


def _pack_bf16(x):
    half = WPAD // 2
    hb = lax.bitcast_convert_type(
        x[:half].astype(jnp.bfloat16), jnp.uint16).astype(jnp.uint32)
    lb = lax.bitcast_convert_type(
        x[half:].astype(jnp.bfloat16), jnp.uint16).astype(jnp.uint32)
    return lax.bitcast_convert_type((hb << 16) | lb, jnp.int32)


def _build_body(inp_ref, stnT_ref, afc_ref, xP_ref, sfcT_ref):
    x0T = jnp.concatenate(
        [inp_ref[...], stnT_ref[...],
         jnp.zeros((WPAD - W528, BLK), jnp.float32)], axis=0)
    xP_ref[...] = _pack_bf16(x0T)
    _acc_sigmoid(sfcT_ref, jnp.dot(x0T, afc_ref[...],
                                   preferred_element_type=jnp.float32))


def _build(inp, stnT, afc):
    return pl.pallas_call(
        _build_body,
        grid=(NPD // BLK,),
        in_specs=[
            pl.BlockSpec((B, BLK), lambda i: (0, i)),
            pl.BlockSpec((SW, BLK), lambda i: (0, i)),
            pl.BlockSpec((BLK, C), lambda i: (i, 0)),
        ],
        out_specs=[
            pl.BlockSpec((WPAD // 2, BLK), lambda i: (0, i)),
            pl.BlockSpec((WPAD, C), lambda i: (0, 0)),
        ],
        out_shape=[
            jax.ShapeDtypeStruct((WPAD // 2, NPD), jnp.int32),
            jax.ShapeDtypeStruct((WPAD, C), jnp.float32),
        ],
    )(inp, stnT, afc)


def _x1T(yT_ref, sfcT_ref, afc_ref):
    gT = lax.dot_general(
        sfcT_ref[...], afc_ref[...], (((1,), (1,)), ((), ())),
        preferred_element_type=jnp.float32)
    return yT_ref[...] + gT


def _r1_body(yT_ref, afc_ref, sfcT_ref, inp_ref, stnT_ref,
             wr_ref, wu_ref, brr_ref, bru_ref,
             xPb_ref, uT_ref, sfc2T_ref):
    x1T = _x1T(yT_ref, sfcT_ref, afc_ref)
    rT = jax.nn.sigmoid(
        lax.dot_general(wr_ref[...], x1T, (((0,), (0,)), ((), ())),
                        preferred_element_type=jnp.float32)
        + brr_ref[...])
    uT_ref[...] = jax.nn.sigmoid(
        lax.dot_general(wu_ref[...], x1T, (((0,), (0,)), ((), ())),
                        preferred_element_type=jnp.float32)
        + bru_ref[...])
    st2T = rT * stnT_ref[...]
    x0bT = jnp.concatenate(
        [inp_ref[...], st2T,
         jnp.zeros((WPAD - W528, BLK), jnp.float32)], axis=0)
    xPb_ref[...] = _pack_bf16(x0bT)
    _acc_sigmoid(sfc2T_ref, jnp.dot(x0bT, afc_ref[...],
                                    preferred_element_type=jnp.float32))


def _round1(yT, afc, sfcT, inp, stnT, wr, wu, brr, bru):
    return pl.pallas_call(
        _r1_body,
        grid=(NPD // BLK,),
        in_specs=[
            pl.BlockSpec((WPAD, BLK), lambda i: (0, i)),
            pl.BlockSpec((BLK, C), lambda i: (i, 0)),
            pl.BlockSpec((WPAD, C), lambda i: (0, 0)),
            pl.BlockSpec((B, BLK), lambda i: (0, i)),
            pl.BlockSpec((SW, BLK), lambda i: (0, i)),
            pl.BlockSpec((WPAD, SW), lambda i: (0, 0)),
            pl.BlockSpec((WPAD, SW), lambda i: (0, 0)),
            pl.BlockSpec((SW, 1), lambda i: (0, 0)),
            pl.BlockSpec((SW, 1), lambda i: (0, 0)),
        ],
        out_specs=[
            pl.BlockSpec((WPAD // 2, BLK), lambda i: (0, i)),
            pl.BlockSpec((SW, BLK), lambda i: (0, i)),
            pl.BlockSpec((WPAD, C), lambda i: (0, 0)),
        ],
        out_shape=[
            jax.ShapeDtypeStruct((WPAD // 2, NPD), jnp.int32),
            jax.ShapeDtypeStruct((SW, NPD), jnp.float32),
            jax.ShapeDtypeStruct((WPAD, C), jnp.float32),
        ],
    )(yT, afc, sfcT, inp, stnT, wr, wu, brr, bru)


def _r2_body(yT_ref, afc_ref, sfc2T_ref, stnT_ref, uT_ref,
             wc_ref, bt_ref, nsT_ref):
    x1T = _x1T(yT_ref, sfc2T_ref, afc_ref)
    out2T = lax.dot_general(
        wc_ref[...], x1T, (((0,), (0,)), ((), ())),
        preferred_element_type=jnp.float32) + bt_ref[...]
    cT = jnp.tanh(out2T)
    uT = uT_ref[...]
    nsT_ref[...] = uT * stnT_ref[...] + (1.0 - uT) * cT


def _round2(yT, afc, sfc2T, stnT, uT, wc, bt):
    return pl.pallas_call(
        _r2_body,
        grid=(NPD // BLK,),
        in_specs=[
            pl.BlockSpec((WPAD, BLK), lambda i: (0, i)),
            pl.BlockSpec((BLK, C), lambda i: (i, 0)),
            pl.BlockSpec((WPAD, C), lambda i: (0, 0)),
            pl.BlockSpec((SW, BLK), lambda i: (0, i)),
            pl.BlockSpec((SW, BLK), lambda i: (0, i)),
            pl.BlockSpec((WPAD, SW), lambda i: (0, 0)),
            pl.BlockSpec((SW, 1), lambda i: (0, 0)),
        ],
        out_specs=pl.BlockSpec((SW, BLK), lambda i: (0, i)),
        out_shape=jax.ShapeDtypeStruct((SW, NPD), jnp.float32),
    )(yT, afc, sfc2T, stnT, uT, wc, bt)


def kernel(inputs, state, weights_0, bias_0, weights_1, bias_1, weights_01,
           bias_01, weights_11, bias_11, afc_mx, L_val, L1, L_row, L_col):
    stnT = state.reshape(B, N, NU).transpose(0, 2, 1).reshape(SW, N)
    stnT = jnp.pad(stnT, ((0, 0), (0, NPD - N)))
    inp = jnp.pad(inputs, ((0, 0), (0, NPD - N)))
    afc = jnp.pad(afc_mx, ((0, NPD - N), (0, 0)))

    e = L_val.shape[0]
    epad = ((e + NBUF * KE - 1) // (NBUF * KE)) * (NBUF * KE)
    pad = epad - e
    crp = jnp.pad(L_col * 16384 + L_row, (0, pad))
    valp = jnp.pad(L_val, (0, pad))

    eyeb = jnp.eye(B, dtype=jnp.float32)
    zrows = jnp.zeros((WPAD - W528, SW), jnp.float32)

    def wexp(w):
        return jnp.concatenate(
            [jnp.kron(eyeb, w[:1]), jnp.kron(eyeb, w[1:]), zrows], axis=0)

    wr = wexp(weights_0[:, :NU])
    wu = wexp(weights_0[:, NU:])
    wc = wexp(weights_1)
    brr = jnp.tile(bias_0[:NU], B)[:, None]
    bru = jnp.tile(bias_0[NU:], B)[:, None]
    bt1 = jnp.tile(bias_1, B)[:, None]

    x0T, sfcT = _build(inp, stnT, afc)
    yT1 = _sc_spmm(x0T, crp, valp)
    x0bT, uT, sfc2T = _round1(yT1, afc, sfcT, inp, stnT,
                              wr, wu, brr, bru)

    yT2 = _sc_spmm(x0bT, crp, valp)
    nsT = _round2(yT2, afc, sfc2T, stnT, uT, wc, bt1)

    return nsT[:, :N].reshape(B, NU, N).transpose(0, 2, 1).reshape(B, N * NU)
